# baseline (speedup 1.0000x reference)
import functools

import jax
from jax import lax
from jax.experimental import pallas as pl
from jax.experimental.pallas import tpu as pltpu
from jax.experimental.pallas import tpu_sc as plsc

_CHUNK_ROWS = 32
_TC_ROWS = 1024
_TC_BLOCK = 512


def _make_sc_copy(seq_len, d_model, dtype, sc_rows):
    info = plsc.get_sparse_core_info()
    nc, ns = info.num_cores, info.num_subcores
    nw = nc * ns
    rows_per = sc_rows // nw
    nchunks = rows_per // _CHUNK_ROWS
    n_sp = nchunks // 2
    n_tile = nchunks - n_sp
    mesh = plsc.VectorSubcoreMesh(core_axis_name="c", subcore_axis_name="s")

    scratch = [
        pltpu.VMEM((_CHUNK_ROWS, d_model), dtype),
        pltpu.VMEM((_CHUNK_ROWS, d_model), dtype),
        pltpu.VMEM_SHARED((2 * ns, _CHUNK_ROWS, d_model), dtype),
    ]
    scratch += [pltpu.SemaphoreType.DMA] * 8

    @functools.partial(
        pl.kernel,
        mesh=mesh,
        out_type=jax.ShapeDtypeStruct((seq_len, d_model), dtype),
        scratch_types=scratch,
    )
    def sc_copy(table_hbm, out_hbm, tb0, tb1, shared, *sems):
        tg = sems[0:2]
        ts = sems[2:4]
        sg = sems[4:6]
        ss = sems[6:8]
        sid = lax.axis_index("s")
        wid = lax.axis_index("c") * ns + sid
        base = wid * rows_per
        tbufs = (tb0, tb1)
        tscat = [None, None]
        sscat = [None, None]
        for k in range(max(n_tile, n_sp)):
            b = k % 2
            gt = gs = None
            if k < n_tile:
                lo_t = base + k * _CHUNK_ROWS
                if tscat[b] is not None:
                    tscat[b].wait()
                gt = pltpu.async_copy(
                    table_hbm.at[pl.ds(lo_t, _CHUNK_ROWS)], tbufs[b], tg[b]
                )
            if k < n_sp:
                lo_s = base + (n_tile + k) * _CHUNK_ROWS
                if sscat[b] is not None:
                    sscat[b].wait()
                gs = pltpu.async_copy(
                    table_hbm.at[pl.ds(lo_s, _CHUNK_ROWS)],
                    shared.at[2 * sid + b],
                    sg[b],
                )
            if gt is not None:
                gt.wait()
                tscat[b] = pltpu.async_copy(
                    tbufs[b], out_hbm.at[pl.ds(lo_t, _CHUNK_ROWS)], ts[b]
                )
            if gs is not None:
                gs.wait()
                sscat[b] = pltpu.async_copy(
                    shared.at[2 * sid + b],
                    out_hbm.at[pl.ds(lo_s, _CHUNK_ROWS)],
                    ss[b],
                )
        for b in (0, 1):
            if tscat[b] is not None:
                tscat[b].wait()
            if sscat[b] is not None:
                sscat[b].wait()

    return sc_copy


def _tc_fill_body(table_ref, _, out_ref):
    out_ref[:, :] = table_ref[:, :]


def _tc_fill(table, partial_out, sc_rows):
    seq_len, d_model = partial_out.shape
    nblk = (seq_len - sc_rows) // _TC_BLOCK
    blk0 = sc_rows // _TC_BLOCK
    return pl.pallas_call(
        _tc_fill_body,
        grid=(nblk,),
        in_specs=[
            pl.BlockSpec((_TC_BLOCK, d_model), lambda i: (blk0 + i, 0)),
            pl.BlockSpec(memory_space=pl.ANY),
        ],
        out_specs=pl.BlockSpec((_TC_BLOCK, d_model), lambda i: (blk0 + i, 0)),
        out_shape=jax.ShapeDtypeStruct((seq_len, d_model), partial_out.dtype),
        input_output_aliases={1: 0},
    )(table, partial_out)


def kernel(x, pos_table):
    seq_len = x.shape[1]
    d_model = pos_table.shape[1]
    table = pos_table[:seq_len]
    sc_rows = seq_len - _TC_ROWS
    partial = _make_sc_copy(seq_len, d_model, pos_table.dtype, sc_rows)(table)
    out = _tc_fill(table, partial, sc_rows)
    return out[None]


# speedup vs baseline: 1.0238x; 1.0238x over previous
import functools

import jax
from jax import lax
from jax.experimental import pallas as pl
from jax.experimental.pallas import tpu as pltpu
from jax.experimental.pallas import tpu_sc as plsc

_CHUNK_ROWS = 32


def _make_sc_copy(seq_len, d_model, dtype):
    info = plsc.get_sparse_core_info()
    nc, ns = info.num_cores, info.num_subcores
    nw = nc * ns
    rows_per = seq_len // nw
    nchunks = rows_per // _CHUNK_ROWS
    n_sp = nchunks // 2
    n_tile = nchunks - n_sp
    mesh = plsc.VectorSubcoreMesh(core_axis_name="c", subcore_axis_name="s")

    scratch = [
        pltpu.VMEM((_CHUNK_ROWS, d_model), dtype),
        pltpu.VMEM((_CHUNK_ROWS, d_model), dtype),
        pltpu.VMEM_SHARED((2 * ns, _CHUNK_ROWS, d_model), dtype),
    ]
    scratch += [pltpu.SemaphoreType.DMA] * 8

    @functools.partial(
        pl.kernel,
        mesh=mesh,
        out_type=jax.ShapeDtypeStruct((seq_len, d_model), dtype),
        scratch_types=scratch,
    )
    def sc_copy(table_hbm, out_hbm, tb0, tb1, shared, *sems):
        tg = sems[0:2]
        ts = sems[2:4]
        sg = sems[4:6]
        ss = sems[6:8]
        sid = lax.axis_index("s")
        wid = lax.axis_index("c") * ns + sid
        base = wid * rows_per
        tbufs = (tb0, tb1)
        tscat = [None, None]
        sscat = [None, None]
        for k in range(max(n_tile, n_sp)):
            b = k % 2
            gt = gs = None
            if k < n_tile:
                lo_t = base + k * _CHUNK_ROWS
                if tscat[b] is not None:
                    tscat[b].wait()
                gt = pltpu.async_copy(
                    table_hbm.at[pl.ds(lo_t, _CHUNK_ROWS)], tbufs[b], tg[b]
                )
            if k < n_sp:
                lo_s = base + (n_tile + k) * _CHUNK_ROWS
                if sscat[b] is not None:
                    sscat[b].wait()
                gs = pltpu.async_copy(
                    table_hbm.at[pl.ds(lo_s, _CHUNK_ROWS)],
                    shared.at[2 * sid + b],
                    sg[b],
                )
            if gt is not None:
                gt.wait()
                tscat[b] = pltpu.async_copy(
                    tbufs[b], out_hbm.at[pl.ds(lo_t, _CHUNK_ROWS)], ts[b]
                )
            if gs is not None:
                gs.wait()
                sscat[b] = pltpu.async_copy(
                    shared.at[2 * sid + b],
                    out_hbm.at[pl.ds(lo_s, _CHUNK_ROWS)],
                    ss[b],
                )
        for b in (0, 1):
            if tscat[b] is not None:
                tscat[b].wait()
            if sscat[b] is not None:
                sscat[b].wait()

    return sc_copy


def kernel(x, pos_table):
    seq_len = x.shape[1]
    d_model = pos_table.shape[1]
    table = pos_table[:seq_len]
    out = _make_sc_copy(seq_len, d_model, pos_table.dtype)(table)
    return out[None]
